# baseline (speedup 1.0000x reference)
import jax
import jax.numpy as jnp
from jax.experimental import pallas as pl

_BATCH = 16384
_BITS = 128
_ROWS_PER_BLOCK = 1024

_ROT_A = (13, 15, 26, 6)
_ROT_B = (17, 29, 16, 24)


def _threefry_bits(ctr: jax.Array) -> jax.Array:
    ks0 = jnp.uint32(0)
    ks1 = jnp.uint32(42)
    ks2 = jnp.uint32(0 ^ 42 ^ 0x1BD11BDA)
    ks = (ks0, ks1, ks2)

    x0 = jnp.full(ctr.shape, ks0, jnp.uint32)
    x1 = ctr + ks1

    def rotl(v, d):
        return (v << jnp.uint32(d)) | (v >> jnp.uint32(32 - d))

    for i in range(5):
        for r in (_ROT_A if i % 2 == 0 else _ROT_B):
            x0 = x0 + x1
            x1 = rotl(x1, r)
            x1 = x0 ^ x1
        x0 = x0 + ks[(i + 1) % 3]
        x1 = x1 + ks[(i + 2) % 3] + jnp.uint32(i + 1)
    return x0 ^ x1


def _bits_kernel(x_ref, o_ref):
    p = pl.program_id(0)
    base = jnp.uint32(p * _ROWS_PER_BLOCK * _BITS)
    ctr = base + jax.lax.broadcasted_iota(
        jnp.uint32, (_ROWS_PER_BLOCK, _BITS), 0) * jnp.uint32(_BITS) \
        + jax.lax.broadcasted_iota(jnp.uint32, (_ROWS_PER_BLOCK, _BITS), 1)
    bits = _threefry_bits(ctr)
    u = jax.lax.bitcast_convert_type(
        (bits >> jnp.uint32(9)) | jnp.uint32(0x3F800000), jnp.float32) - 1.0
    o_ref[...] = jnp.where(u < x_ref[...], 1.0, 0.0)


def kernel(x):
    x2 = x.reshape(_BATCH, 1)
    grid = (_BATCH // _ROWS_PER_BLOCK,)
    out = pl.pallas_call(
        _bits_kernel,
        grid=grid,
        in_specs=[pl.BlockSpec((_ROWS_PER_BLOCK, 1), lambda p: (p, 0))],
        out_specs=pl.BlockSpec((_ROWS_PER_BLOCK, _BITS), lambda p: (p, 0)),
        out_shape=jax.ShapeDtypeStruct((_BATCH, _BITS), jnp.float32),
    )(x2)
    return out


# speedup vs baseline: 1.0242x; 1.0242x over previous
import jax
import jax.numpy as jnp
from jax.experimental import pallas as pl

_BATCH = 16384
_BITS = 128
_ROWS_PER_BLOCK = 1024

_ROT_A = (13, 15, 26, 6)
_ROT_B = (17, 29, 16, 24)


def _threefry_bits(x1: jax.Array) -> jax.Array:
    ks = (jnp.uint32(0), jnp.uint32(42), jnp.uint32(0 ^ 42 ^ 0x1BD11BDA))

    def rotl(v, d):
        return (v << jnp.uint32(d)) | (v >> jnp.uint32(32 - d))

    x0 = x1
    x1 = x0 ^ rotl(x1, _ROT_A[0])
    for r in _ROT_A[1:]:
        x0 = x0 + x1
        x1 = rotl(x1, r)
        x1 = x0 ^ x1
    x0 = x0 + ks[1]
    x1 = x1 + (ks[2] + jnp.uint32(1))
    for i in range(1, 5):
        for r in (_ROT_A if i % 2 == 0 else _ROT_B):
            x0 = x0 + x1
            x1 = rotl(x1, r)
            x1 = x0 ^ x1
        x0 = x0 + ks[(i + 1) % 3]
        x1 = x1 + (ks[(i + 2) % 3] + jnp.uint32(i + 1))
    return x0 ^ x1


def _bits_kernel(x_ref, o_ref):
    p = pl.program_id(0)
    shape = (_ROWS_PER_BLOCK, _BITS)
    base = (p * _ROWS_PER_BLOCK * _BITS + 42).astype(jnp.uint32)
    x1 = base + (
        jax.lax.broadcasted_iota(jnp.uint32, shape, 0) * jnp.uint32(_BITS)
        + jax.lax.broadcasted_iota(jnp.uint32, shape, 1))
    bits = _threefry_bits(x1)
    u = jax.lax.bitcast_convert_type(
        (bits >> jnp.uint32(9)) | jnp.uint32(0x3F800000), jnp.float32) - 1.0
    o_ref[...] = jnp.where(u < x_ref[...], 1.0, 0.0)


def kernel(x):
    x2 = x.reshape(_BATCH, 1)
    grid = (_BATCH // _ROWS_PER_BLOCK,)
    out = pl.pallas_call(
        _bits_kernel,
        grid=grid,
        in_specs=[pl.BlockSpec((_ROWS_PER_BLOCK, 1), lambda p: (p, 0))],
        out_specs=pl.BlockSpec((_ROWS_PER_BLOCK, _BITS), lambda p: (p, 0)),
        out_shape=jax.ShapeDtypeStruct((_BATCH, _BITS), jnp.float32),
    )(x2)
    return out
